# baseline (speedup 1.0000x reference)
import functools

import jax
import jax.numpy as jnp
from jax.experimental import pallas as pl
from jax.experimental.pallas import tpu as pltpu
from jax.experimental.pallas import tpu_sc as plsc

B = 2
N = 2048
K = 16

_HI = jax.lax.Precision.DEFAULT


def _dot(a, b):
    return jax.lax.dot_general(
        a, b, (((1,), (1,)), ((), ())), precision=_HI,
        preferred_element_type=jnp.float32)


def _lrelu(x):
    return jnp.where(x >= 0, x, 0.2 * x)


def _stn_body(x_ref, w1, b1, w2, b2, w3, b3, fw1, fb1, fw2, fb2, fw3, fb3,
              out_ref):
    X = x_ref[...]

    def convbn(h, w, b):
        y = _dot(h, w[...]) + b[...]
        m = jnp.mean(y, axis=0, keepdims=True)
        v = jnp.mean((y - m) ** 2, axis=0, keepdims=True)
        return jnp.maximum((y - m) / jnp.sqrt(v + 1e-5), 0.0)

    h = convbn(X, w1, b1)
    h = convbn(h, w2, b2)
    h = convbn(h, w3, b3)
    mx = jnp.concatenate(
        [jnp.max(h[b * N:(b + 1) * N], axis=0, keepdims=True)
         for b in range(B)], axis=0)
    f = jnp.maximum(_dot(mx, fw1[...]) + fb1[...], 0.0)
    f = jnp.maximum(_dot(f, fw2[...]) + fb2[...], 0.0)
    f = _dot(f, fw3[...]) + fb3[...]
    iot = jax.lax.broadcasted_iota(jnp.int32, (1, 144), 1)
    f = f + jnp.where(iot % 13 == 0, 1.0, 0.0)
    for b in range(B):
        Tb = jnp.concatenate(
            [f[b:b + 1, 12 * j:12 * (j + 1)] for j in range(12)], axis=0)
        out_ref[b * N:(b + 1) * N, :] = jax.lax.dot_general(
            X[b * N:(b + 1) * N], Tb, (((1,), (0,)), ((), ())), precision=_HI,
            preferred_element_type=jnp.float32)


def _stn(A, p):
    X = A.reshape(B * N, 12)
    r2 = lambda t: t.reshape(1, -1)
    args = (X, p['w1'], r2(p['b1']), p['w2'], r2(p['b2']), p['w3'],
            r2(p['b3']), p['fw1'], r2(p['fb1']), p['fw2'], r2(p['fb2']),
            p['fw3'], r2(p['fb3']))
    out = pl.pallas_call(
        _stn_body,
        out_shape=jax.ShapeDtypeStruct((B * N, 12), jnp.float32),
    )(*args)
    return out.reshape(B, N, 12)


_KNN_BLK = 256


def _knn_body(a_ref, at_ref, out_ref):
    a = a_ref[0]
    at = at_ref[0]
    d = jax.lax.dot_general(a, at, (((1,), (0,)), ((), ())), precision=_HI,
                            preferred_element_type=jnp.float32)
    xx_r = jnp.sum(a * a, axis=1, keepdims=True)
    xx_c = jnp.sum(at * at, axis=0, keepdims=True)
    pd = 2.0 * d - xx_r - xx_c
    iota = jax.lax.broadcasted_iota(jnp.int32, (_KNN_BLK, N), 1)
    cols = []
    for j in range(K + 1):
        m = jnp.max(pd, axis=1, keepdims=True)
        amin = jnp.min(jnp.where(pd == m, iota, N), axis=1, keepdims=True)
        if j > 0:
            cols.append(amin)
        pd = jnp.where(iota == amin, -jnp.inf, pd)
    out_ref[0] = jnp.concatenate(cols, axis=1)


def _knn(A):
    C = A.shape[2]
    At = A.transpose(0, 2, 1)
    return pl.pallas_call(
        _knn_body,
        grid=(B, N // _KNN_BLK),
        in_specs=[
            pl.BlockSpec((1, _KNN_BLK, C), lambda b, i: (b, i, 0)),
            pl.BlockSpec((1, C, N), lambda b, i: (b, 0, 0)),
        ],
        out_specs=pl.BlockSpec((1, _KNN_BLK, K), lambda b, i: (b, i, 0)),
        out_shape=jax.ShapeDtypeStruct((B, N, K), jnp.int32),
    )(A, At)


def _sc_gather(S, gidx):
    NI = gidx.shape[0]
    C3 = S.shape[1]
    W = 128
    mesh = plsc.VectorSubcoreMesh(core_axis_name="core",
                                  subcore_axis_name="subcore")
    idx2 = gidx.reshape(1, NI)

    @functools.partial(
        pl.kernel,
        out_type=jax.ShapeDtypeStruct((NI, C3), S.dtype),
        mesh=mesh)
    def k(x_hbm, i_hbm, o_hbm):
        def body(i_vmem, o_vmem):
            pltpu.sync_copy(x_hbm.at[i_vmem.at[0]], o_vmem)

        pltpu.emit_pipeline(
            body,
            grid=(NI // W,),
            in_specs=[pl.BlockSpec((1, W), index_map=lambda i: (0, i))],
            out_specs=[pl.BlockSpec((W, C3), index_map=lambda i: (i, 0))],
            core_axis_name=('core', 'subcore'),
            dimension_semantics=(pltpu.PARALLEL,),
        )(i_hbm, o_hbm)

    return k(S, idx2)


_LYR_BLK = 128


def _layer_a_body(C, Cs, O, g_ref, ac_ref, an_ref, scr_ref,
                  wcn_ref, wcc_ref, wnn_ref, wnc_ref, wa1_ref, wa2_ref,
                  fc_ref, e_ref, mn_ref, st_ref):
    first = jnp.logical_and(pl.program_id(0) == 0, pl.program_id(1) == 0)
    R = _LYR_BLK * K

    g = g_ref[...]
    coor_nb = g[:, 0:C]
    nor_nb = g[:, Cs:Cs + C]
    x_nb = g[:, 2 * Cs:2 * Cs + C]
    ac = ac_ref[0]
    an = an_ref[0]
    scr = scr_ref[0]

    def bcast(cen):
        return jnp.broadcast_to(cen[:, None, :], (_LYR_BLK, K, C)).reshape(R, C)

    fc = _dot(jnp.concatenate([coor_nb, bcast(ac)], axis=1),
              jnp.concatenate([wcn_ref[...], wcc_ref[...]], axis=1))
    fn = _dot(jnp.concatenate([nor_nb, bcast(an)], axis=1),
              jnp.concatenate([wnn_ref[...], wnc_ref[...]], axis=1))
    e = _dot(jnp.concatenate([bcast(scr) - x_nb, x_nb], axis=1),
             jnp.concatenate([wa1_ref[...], wa2_ref[...]], axis=1))

    fc_ref[...] = fc
    e_ref[...] = e
    mn_ref[0] = jnp.max(fn.reshape(_LYR_BLK, K, O), axis=1)

    @pl.when(first)
    def _():
        st_ref[...] = jnp.zeros_like(st_ref)

    def rs(t):
        return jnp.sum(t, axis=0, keepdims=True)

    st_ref[0:6] += jnp.concatenate(
        [rs(fc), rs(fn), rs(e), rs(fc * fc), rs(fn * fn), rs(e * e)], axis=0)


def _layer_b_body(O, fc_ref, e_ref, mn_ref, aff_ref, co_ref, no_ref):
    aff = aff_ref[...]
    fcn = _lrelu((fc_ref[...].reshape(_LYR_BLK, K, O) - aff[0:1]) * aff[3:4])
    en = _lrelu((e_ref[...].reshape(_LYR_BLK, K, O) - aff[2:3]) * aff[5:6])
    emax = jnp.max(en, axis=1, keepdims=True)
    ex = jnp.exp(en - emax)
    att = ex / jnp.sum(ex, axis=1, keepdims=True)
    co_ref[0] = jnp.sum(att * fcn, axis=1)
    no_ref[0] = _lrelu((mn_ref[0] - aff[1:2]) * aff[4:5])


def _layer(G, Ac, An, scr, Wc, Wn, Wa, C, Cs):
    O = Wc.shape[0]
    C3p = G.shape[1]
    NB = N // _LYR_BLK
    fc, e, mn, st = pl.pallas_call(
        functools.partial(_layer_a_body, C, Cs, O),
        grid=(B, NB),
        in_specs=[
            pl.BlockSpec((_LYR_BLK * K, C3p), lambda b, i: (b * NB + i, 0)),
            pl.BlockSpec((1, _LYR_BLK, C), lambda b, i: (b, i, 0)),
            pl.BlockSpec((1, _LYR_BLK, C), lambda b, i: (b, i, 0)),
            pl.BlockSpec((1, _LYR_BLK, C), lambda b, i: (b, i, 0)),
            pl.BlockSpec((O, C), lambda b, i: (0, 0)),
            pl.BlockSpec((O, C), lambda b, i: (0, 0)),
            pl.BlockSpec((O, C), lambda b, i: (0, 0)),
            pl.BlockSpec((O, C), lambda b, i: (0, 0)),
            pl.BlockSpec((O, C), lambda b, i: (0, 0)),
            pl.BlockSpec((O, C), lambda b, i: (0, 0)),
        ],
        out_specs=[
            pl.BlockSpec((_LYR_BLK * K, O), lambda b, i: (b * NB + i, 0)),
            pl.BlockSpec((_LYR_BLK * K, O), lambda b, i: (b * NB + i, 0)),
            pl.BlockSpec((1, _LYR_BLK, O), lambda b, i: (b, i, 0)),
            pl.BlockSpec((8, O), lambda b, i: (0, 0)),
        ],
        out_shape=[
            jax.ShapeDtypeStruct((B * N * K, O), jnp.float32),
            jax.ShapeDtypeStruct((B * N * K, O), jnp.float32),
            jax.ShapeDtypeStruct((B, N, O), jnp.float32),
            jax.ShapeDtypeStruct((8, O), jnp.float32),
        ],
    )(G, Ac, An, scr, Wc[:, :C], Wc[:, C:], Wn[:, :C],
      Wn[:, C:], Wa[:, :C], Wa[:, C:])

    cnt = float(B * N * K)
    m = st[0:3] / cnt
    v = st[3:6] / cnt - m * m
    aff = jnp.concatenate([m, 1.0 / jnp.sqrt(v + 1e-5)], axis=0)

    co, no = pl.pallas_call(
        functools.partial(_layer_b_body, O),
        grid=(B, NB),
        in_specs=[
            pl.BlockSpec((_LYR_BLK * K, O), lambda b, i: (b * NB + i, 0)),
            pl.BlockSpec((_LYR_BLK * K, O), lambda b, i: (b * NB + i, 0)),
            pl.BlockSpec((1, _LYR_BLK, O), lambda b, i: (b, i, 0)),
            pl.BlockSpec((8, O), lambda b, i: (0, 0)),
        ],
        out_specs=[
            pl.BlockSpec((1, _LYR_BLK, O), lambda b, i: (b, i, 0)),
            pl.BlockSpec((1, _LYR_BLK, O), lambda b, i: (b, i, 0)),
        ],
        out_shape=[
            jax.ShapeDtypeStruct((B, N, O), jnp.float32),
            jax.ShapeDtypeStruct((B, N, O), jnp.float32),
        ],
    )(fc, e, mn, aff)
    return co, no


_HEAD_BLK = 512
_R = B * N


def _affconv_body(apply_aff, x_ref, aff_ref, w_ref, y_ref, st_ref):
    i = pl.program_id(0)
    x = x_ref[...]
    if apply_aff:
        aff = aff_ref[...]
        x = _lrelu((x - aff[0:1]) * aff[1:2])
    y = _dot(x, w_ref[...])
    y_ref[...] = y

    @pl.when(i == 0)
    def _():
        st_ref[...] = jnp.zeros_like(st_ref)

    st_ref[0:1] += jnp.sum(y, axis=0, keepdims=True)
    st_ref[1:2] += jnp.sum(y * y, axis=0, keepdims=True)


def _affconv(x, aff, w):
    O, C = w.shape
    apply_aff = aff is not None
    if aff is None:
        aff = jnp.zeros((2, C), jnp.float32)
    y, st = pl.pallas_call(
        functools.partial(_affconv_body, apply_aff),
        grid=(_R // _HEAD_BLK,),
        in_specs=[
            pl.BlockSpec((_HEAD_BLK, C), lambda i: (i, 0)),
            pl.BlockSpec((2, C), lambda i: (0, 0)),
            pl.BlockSpec((O, C), lambda i: (0, 0)),
        ],
        out_specs=[
            pl.BlockSpec((_HEAD_BLK, O), lambda i: (i, 0)),
            pl.BlockSpec((2, O), lambda i: (0, 0)),
        ],
        out_shape=[
            jax.ShapeDtypeStruct((_R, O), jnp.float32),
            jax.ShapeDtypeStruct((2, O), jnp.float32),
        ],
    )(x, aff, w)
    return y, st


def _finalize_aff(st):
    m = st[0:1] / _R
    v = st[1:2] / _R - m * m
    return jnp.concatenate([m, 1.0 / jnp.sqrt(v + 1e-5)], axis=0)


def _head_fuse_body(yc_ref, affc_ref, yn_ref, affn_ref, out_ref):
    ac = affc_ref[...]
    an = affn_ref[...]
    coorf = _lrelu((yc_ref[...] - ac[0:1]) * ac[1:2])
    norf = _lrelu((yn_ref[...] - an[0:1]) * an[1:2])
    avg_c = jnp.sum(coorf, axis=1, keepdims=True) / 512.0
    avg_n = jnp.sum(norf, axis=1, keepdims=True) / 512.0
    avg = avg_c + avg_n
    out_ref[:, 0:512] = coorf * (avg_c / avg)
    out_ref[:, 512:1024] = norf * (avg_n / avg)


def _final_body(y_ref, aff_ref, w_ref, out_ref):
    aff = aff_ref[...]
    h = _lrelu((y_ref[...] - aff[0:1]) * aff[1:2])
    s = _dot(h, w_ref[...])
    m = jnp.max(s, axis=1, keepdims=True)
    sh = s - m
    out_ref[...] = sh - jnp.log(jnp.sum(jnp.exp(sh), axis=1, keepdims=True))


def kernel(x, params):
    p = params
    Ac = _stn(x[:, :12, :].transpose(0, 2, 1), p['stn_c'])
    An = _stn(x[:, 12:, :].transpose(0, 2, 1), p['stn_n'])

    offs = (jnp.arange(B, dtype=jnp.int32) * N)[:, None, None]
    couts, nouts = [], []
    layer_ps = [(p['conv1_c'], p['conv1_n'], p['att1']),
                (p['conv2_c'], p['conv2_n'], p['att2']),
                (p['conv3_c'], p['conv3_n'], p['att3'])]
    for Wc, Wn, Wa in layer_ps:
        C = Ac.shape[2]
        Cs = 16 if C < 16 else C
        scr = Ac.transpose(0, 2, 1).reshape(B, N, C)
        parts = [Ac, An, scr]
        if Cs != C:
            zpad = jnp.zeros((B, N, Cs - C), jnp.float32)
            parts = [Ac, zpad, An, zpad, scr, zpad]
        C3p = 3 * Cs
        if C3p % 128:
            tail = 128 - C3p % 128
            parts.append(jnp.zeros((B, N, tail), jnp.float32))
            C3p += tail
        S = jnp.concatenate(parts, axis=2).reshape(B * N, C3p)
        idx = _knn(Ac)
        gidx = (idx + offs).reshape(B * N * K)
        G = _sc_gather(S, gidx)
        Ac, An = _layer(G, Ac, An, scr, Wc, Wn, Wa, C, Cs)
        couts.append(Ac)
        nouts.append(An)

    Xc = jnp.concatenate(couts, axis=2).reshape(B * N, 448)
    Xn = jnp.concatenate(nouts, axis=2).reshape(B * N, 448)
    yc, stc = _affconv(Xc, None, p['conv4_c'])
    yn, stn = _affconv(Xn, None, p['conv4_n'])
    h = pl.pallas_call(
        _head_fuse_body,
        grid=(_R // _HEAD_BLK,),
        in_specs=[
            pl.BlockSpec((_HEAD_BLK, 512), lambda i: (i, 0)),
            pl.BlockSpec((2, 512), lambda i: (0, 0)),
            pl.BlockSpec((_HEAD_BLK, 512), lambda i: (i, 0)),
            pl.BlockSpec((2, 512), lambda i: (0, 0)),
        ],
        out_specs=pl.BlockSpec((_HEAD_BLK, 1024), lambda i: (i, 0)),
        out_shape=jax.ShapeDtypeStruct((_R, 1024), jnp.float32),
    )(yc, _finalize_aff(stc), yn, _finalize_aff(stn))
    y1, st1 = _affconv(h, None, p['fa'])
    y2, st2 = _affconv(y1, _finalize_aff(st1), p['pred1'])
    y3, st3 = _affconv(y2, _finalize_aff(st2), p['pred2'])
    y4, st4 = _affconv(y3, _finalize_aff(st3), p['pred3'])
    out = pl.pallas_call(
        _final_body,
        grid=(_R // _HEAD_BLK,),
        in_specs=[
            pl.BlockSpec((_HEAD_BLK, 128), lambda i: (i, 0)),
            pl.BlockSpec((2, 128), lambda i: (0, 0)),
            pl.BlockSpec((15, 128), lambda i: (0, 0)),
        ],
        out_specs=pl.BlockSpec((_HEAD_BLK, 15), lambda i: (i, 0)),
        out_shape=jax.ShapeDtypeStruct((_R, 15), jnp.float32),
    )(y4, _finalize_aff(st4), p['pred4'])
    return out.reshape(B, N, 15)


# speedup vs baseline: 1.0664x; 1.0664x over previous
import functools

import jax
import jax.numpy as jnp
from jax.experimental import pallas as pl
from jax.experimental.pallas import tpu as pltpu
from jax.experimental.pallas import tpu_sc as plsc

B = 2
N = 2048
K = 16

_HI = jax.lax.Precision.DEFAULT


def _dot(a, b):
    return jax.lax.dot_general(
        a, b, (((1,), (1,)), ((), ())), precision=_HI,
        preferred_element_type=jnp.float32)


def _lrelu(x):
    return jnp.where(x >= 0, x, 0.2 * x)


def _stn_body(x_ref, w1, b1, w2, b2, w3, b3, fw1, fb1, fw2, fb2, fw3, fb3,
              out_ref):
    X = x_ref[...]

    def convbn(h, w, b):
        y = _dot(h, w[...]) + b[...]
        m = jnp.mean(y, axis=0, keepdims=True)
        v = jnp.mean((y - m) ** 2, axis=0, keepdims=True)
        return jnp.maximum((y - m) / jnp.sqrt(v + 1e-5), 0.0)

    h = convbn(X, w1, b1)
    h = convbn(h, w2, b2)
    h = convbn(h, w3, b3)
    mx = jnp.concatenate(
        [jnp.max(h[b * N:(b + 1) * N], axis=0, keepdims=True)
         for b in range(B)], axis=0)
    f = jnp.maximum(_dot(mx, fw1[...]) + fb1[...], 0.0)
    f = jnp.maximum(_dot(f, fw2[...]) + fb2[...], 0.0)
    f = _dot(f, fw3[...]) + fb3[...]
    iot = jax.lax.broadcasted_iota(jnp.int32, (1, 144), 1)
    f = f + jnp.where(iot % 13 == 0, 1.0, 0.0)
    for b in range(B):
        Tb = jnp.concatenate(
            [f[b:b + 1, 12 * j:12 * (j + 1)] for j in range(12)], axis=0)
        out_ref[b * N:(b + 1) * N, :] = jax.lax.dot_general(
            X[b * N:(b + 1) * N], Tb, (((1,), (0,)), ((), ())), precision=_HI,
            preferred_element_type=jnp.float32)


def _stn(A, p):
    X = A.reshape(B * N, 12)
    r2 = lambda t: t.reshape(1, -1)
    args = (X, p['w1'], r2(p['b1']), p['w2'], r2(p['b2']), p['w3'],
            r2(p['b3']), p['fw1'], r2(p['fb1']), p['fw2'], r2(p['fb2']),
            p['fw3'], r2(p['fb3']))
    out = pl.pallas_call(
        _stn_body,
        out_shape=jax.ShapeDtypeStruct((B * N, 12), jnp.float32),
    )(*args)
    return out.reshape(B, N, 12)


_KNN_BLK = 512


def _knn_body(a_ref, at_ref, out_ref):
    a = a_ref[0]
    at = at_ref[0]
    d = jax.lax.dot_general(a, at, (((1,), (0,)), ((), ())), precision=_HI,
                            preferred_element_type=jnp.float32)
    xx_r = jnp.sum(a * a, axis=1, keepdims=True)
    xx_c = jnp.sum(at * at, axis=0, keepdims=True)
    pd = 2.0 * d - xx_r - xx_c
    iota = jax.lax.broadcasted_iota(jnp.int32, (_KNN_BLK, N), 1)
    cols = []
    for j in range(K + 1):
        m = jnp.max(pd, axis=1, keepdims=True)
        amin = jnp.min(jnp.where(pd == m, iota, N), axis=1, keepdims=True)
        if j > 0:
            cols.append(amin)
        pd = jnp.where(iota == amin, -jnp.inf, pd)
    out_ref[0] = jnp.concatenate(cols, axis=1)


def _knn(A):
    C = A.shape[2]
    At = A.transpose(0, 2, 1)
    return pl.pallas_call(
        _knn_body,
        grid=(B, N // _KNN_BLK),
        in_specs=[
            pl.BlockSpec((1, _KNN_BLK, C), lambda b, i: (b, i, 0)),
            pl.BlockSpec((1, C, N), lambda b, i: (b, 0, 0)),
        ],
        out_specs=pl.BlockSpec((1, _KNN_BLK, K), lambda b, i: (b, i, 0)),
        out_shape=jax.ShapeDtypeStruct((B, N, K), jnp.int32),
    )(A, At)


def _sc_gather(S, gidx):
    NI = gidx.shape[0]
    C3 = S.shape[1]
    W = 128
    mesh = plsc.VectorSubcoreMesh(core_axis_name="core",
                                  subcore_axis_name="subcore")
    idx2 = gidx.reshape(1, NI)

    @functools.partial(
        pl.kernel,
        out_type=jax.ShapeDtypeStruct((NI, C3), S.dtype),
        mesh=mesh)
    def k(x_hbm, i_hbm, o_hbm):
        def body(i_vmem, o_vmem):
            pltpu.sync_copy(x_hbm.at[i_vmem.at[0]], o_vmem)

        pltpu.emit_pipeline(
            body,
            grid=(NI // W,),
            in_specs=[pl.BlockSpec((1, W), index_map=lambda i: (0, i))],
            out_specs=[pl.BlockSpec((W, C3), index_map=lambda i: (i, 0))],
            core_axis_name=('core', 'subcore'),
            dimension_semantics=(pltpu.PARALLEL,),
        )(i_hbm, o_hbm)

    return k(S, idx2)


_LYR_BLK = 256


def _layer_a_body(C, Cs, O, g_ref, ac_ref, an_ref, scr_ref,
                  wcn_ref, wcc_ref, wnn_ref, wnc_ref, wa1_ref, wa2_ref,
                  fc_ref, e_ref, mn_ref, st_ref):
    first = jnp.logical_and(pl.program_id(0) == 0, pl.program_id(1) == 0)
    R = _LYR_BLK * K

    g = g_ref[...]
    coor_nb = g[:, 0:C]
    nor_nb = g[:, Cs:Cs + C]
    x_nb = g[:, 2 * Cs:2 * Cs + C]
    ac = ac_ref[0]
    an = an_ref[0]
    scr = scr_ref[0]

    def bcast(cen):
        return jnp.broadcast_to(cen[:, None, :], (_LYR_BLK, K, C)).reshape(R, C)

    fc = _dot(jnp.concatenate([coor_nb, bcast(ac)], axis=1),
              jnp.concatenate([wcn_ref[...], wcc_ref[...]], axis=1))
    fn = _dot(jnp.concatenate([nor_nb, bcast(an)], axis=1),
              jnp.concatenate([wnn_ref[...], wnc_ref[...]], axis=1))
    e = _dot(jnp.concatenate([bcast(scr) - x_nb, x_nb], axis=1),
             jnp.concatenate([wa1_ref[...], wa2_ref[...]], axis=1))

    fc_ref[...] = fc
    e_ref[...] = e
    mn_ref[0] = jnp.max(fn.reshape(_LYR_BLK, K, O), axis=1)

    @pl.when(first)
    def _():
        st_ref[...] = jnp.zeros_like(st_ref)

    def rs(t):
        return jnp.sum(t, axis=0, keepdims=True)

    st_ref[0:6] += jnp.concatenate(
        [rs(fc), rs(fn), rs(e), rs(fc * fc), rs(fn * fn), rs(e * e)], axis=0)


def _layer_b_body(O, fc_ref, e_ref, mn_ref, aff_ref, co_ref, no_ref):
    aff = aff_ref[...]
    fcn = _lrelu((fc_ref[...].reshape(_LYR_BLK, K, O) - aff[0:1]) * aff[3:4])
    en = _lrelu((e_ref[...].reshape(_LYR_BLK, K, O) - aff[2:3]) * aff[5:6])
    emax = jnp.max(en, axis=1, keepdims=True)
    ex = jnp.exp(en - emax)
    att = ex / jnp.sum(ex, axis=1, keepdims=True)
    co_ref[0] = jnp.sum(att * fcn, axis=1)
    no_ref[0] = _lrelu((mn_ref[0] - aff[1:2]) * aff[4:5])


def _layer(G, Ac, An, scr, Wc, Wn, Wa, C, Cs):
    O = Wc.shape[0]
    C3p = G.shape[1]
    NB = N // _LYR_BLK
    fc, e, mn, st = pl.pallas_call(
        functools.partial(_layer_a_body, C, Cs, O),
        grid=(B, NB),
        in_specs=[
            pl.BlockSpec((_LYR_BLK * K, C3p), lambda b, i: (b * NB + i, 0)),
            pl.BlockSpec((1, _LYR_BLK, C), lambda b, i: (b, i, 0)),
            pl.BlockSpec((1, _LYR_BLK, C), lambda b, i: (b, i, 0)),
            pl.BlockSpec((1, _LYR_BLK, C), lambda b, i: (b, i, 0)),
            pl.BlockSpec((O, C), lambda b, i: (0, 0)),
            pl.BlockSpec((O, C), lambda b, i: (0, 0)),
            pl.BlockSpec((O, C), lambda b, i: (0, 0)),
            pl.BlockSpec((O, C), lambda b, i: (0, 0)),
            pl.BlockSpec((O, C), lambda b, i: (0, 0)),
            pl.BlockSpec((O, C), lambda b, i: (0, 0)),
        ],
        out_specs=[
            pl.BlockSpec((_LYR_BLK * K, O), lambda b, i: (b * NB + i, 0)),
            pl.BlockSpec((_LYR_BLK * K, O), lambda b, i: (b * NB + i, 0)),
            pl.BlockSpec((1, _LYR_BLK, O), lambda b, i: (b, i, 0)),
            pl.BlockSpec((8, O), lambda b, i: (0, 0)),
        ],
        out_shape=[
            jax.ShapeDtypeStruct((B * N * K, O), jnp.float32),
            jax.ShapeDtypeStruct((B * N * K, O), jnp.float32),
            jax.ShapeDtypeStruct((B, N, O), jnp.float32),
            jax.ShapeDtypeStruct((8, O), jnp.float32),
        ],
    )(G, Ac, An, scr, Wc[:, :C], Wc[:, C:], Wn[:, :C],
      Wn[:, C:], Wa[:, :C], Wa[:, C:])

    cnt = float(B * N * K)
    m = st[0:3] / cnt
    v = st[3:6] / cnt - m * m
    aff = jnp.concatenate([m, 1.0 / jnp.sqrt(v + 1e-5)], axis=0)

    co, no = pl.pallas_call(
        functools.partial(_layer_b_body, O),
        grid=(B, NB),
        in_specs=[
            pl.BlockSpec((_LYR_BLK * K, O), lambda b, i: (b * NB + i, 0)),
            pl.BlockSpec((_LYR_BLK * K, O), lambda b, i: (b * NB + i, 0)),
            pl.BlockSpec((1, _LYR_BLK, O), lambda b, i: (b, i, 0)),
            pl.BlockSpec((8, O), lambda b, i: (0, 0)),
        ],
        out_specs=[
            pl.BlockSpec((1, _LYR_BLK, O), lambda b, i: (b, i, 0)),
            pl.BlockSpec((1, _LYR_BLK, O), lambda b, i: (b, i, 0)),
        ],
        out_shape=[
            jax.ShapeDtypeStruct((B, N, O), jnp.float32),
            jax.ShapeDtypeStruct((B, N, O), jnp.float32),
        ],
    )(fc, e, mn, aff)
    return co, no


_HEAD_BLK = 512
_R = B * N


def _affconv_body(apply_aff, x_ref, aff_ref, w_ref, y_ref, st_ref):
    i = pl.program_id(0)
    x = x_ref[...]
    if apply_aff:
        aff = aff_ref[...]
        x = _lrelu((x - aff[0:1]) * aff[1:2])
    y = _dot(x, w_ref[...])
    y_ref[...] = y

    @pl.when(i == 0)
    def _():
        st_ref[...] = jnp.zeros_like(st_ref)

    st_ref[0:1] += jnp.sum(y, axis=0, keepdims=True)
    st_ref[1:2] += jnp.sum(y * y, axis=0, keepdims=True)


def _affconv(x, aff, w):
    O, C = w.shape
    apply_aff = aff is not None
    if aff is None:
        aff = jnp.zeros((2, C), jnp.float32)
    y, st = pl.pallas_call(
        functools.partial(_affconv_body, apply_aff),
        grid=(_R // _HEAD_BLK,),
        in_specs=[
            pl.BlockSpec((_HEAD_BLK, C), lambda i: (i, 0)),
            pl.BlockSpec((2, C), lambda i: (0, 0)),
            pl.BlockSpec((O, C), lambda i: (0, 0)),
        ],
        out_specs=[
            pl.BlockSpec((_HEAD_BLK, O), lambda i: (i, 0)),
            pl.BlockSpec((2, O), lambda i: (0, 0)),
        ],
        out_shape=[
            jax.ShapeDtypeStruct((_R, O), jnp.float32),
            jax.ShapeDtypeStruct((2, O), jnp.float32),
        ],
    )(x, aff, w)
    return y, st


def _finalize_aff(st):
    m = st[0:1] / _R
    v = st[1:2] / _R - m * m
    return jnp.concatenate([m, 1.0 / jnp.sqrt(v + 1e-5)], axis=0)


def _head_fuse_body(yc_ref, affc_ref, yn_ref, affn_ref, out_ref):
    ac = affc_ref[...]
    an = affn_ref[...]
    coorf = _lrelu((yc_ref[...] - ac[0:1]) * ac[1:2])
    norf = _lrelu((yn_ref[...] - an[0:1]) * an[1:2])
    avg_c = jnp.sum(coorf, axis=1, keepdims=True) / 512.0
    avg_n = jnp.sum(norf, axis=1, keepdims=True) / 512.0
    avg = avg_c + avg_n
    out_ref[:, 0:512] = coorf * (avg_c / avg)
    out_ref[:, 512:1024] = norf * (avg_n / avg)


def _final_body(y_ref, aff_ref, w_ref, out_ref):
    aff = aff_ref[...]
    h = _lrelu((y_ref[...] - aff[0:1]) * aff[1:2])
    s = _dot(h, w_ref[...])
    m = jnp.max(s, axis=1, keepdims=True)
    sh = s - m
    out_ref[...] = sh - jnp.log(jnp.sum(jnp.exp(sh), axis=1, keepdims=True))


def kernel(x, params):
    p = params
    Ac = _stn(x[:, :12, :].transpose(0, 2, 1), p['stn_c'])
    An = _stn(x[:, 12:, :].transpose(0, 2, 1), p['stn_n'])

    offs = (jnp.arange(B, dtype=jnp.int32) * N)[:, None, None]
    couts, nouts = [], []
    layer_ps = [(p['conv1_c'], p['conv1_n'], p['att1']),
                (p['conv2_c'], p['conv2_n'], p['att2']),
                (p['conv3_c'], p['conv3_n'], p['att3'])]
    for Wc, Wn, Wa in layer_ps:
        C = Ac.shape[2]
        Cs = 16 if C < 16 else C
        scr = Ac.transpose(0, 2, 1).reshape(B, N, C)
        parts = [Ac, An, scr]
        if Cs != C:
            zpad = jnp.zeros((B, N, Cs - C), jnp.float32)
            parts = [Ac, zpad, An, zpad, scr, zpad]
        C3p = 3 * Cs
        if C3p % 128:
            tail = 128 - C3p % 128
            parts.append(jnp.zeros((B, N, tail), jnp.float32))
            C3p += tail
        S = jnp.concatenate(parts, axis=2).reshape(B * N, C3p)
        idx = _knn(Ac)
        gidx = (idx + offs).reshape(B * N * K)
        G = _sc_gather(S, gidx)
        Ac, An = _layer(G, Ac, An, scr, Wc, Wn, Wa, C, Cs)
        couts.append(Ac)
        nouts.append(An)

    Xc = jnp.concatenate(couts, axis=2).reshape(B * N, 448)
    Xn = jnp.concatenate(nouts, axis=2).reshape(B * N, 448)
    yc, stc = _affconv(Xc, None, p['conv4_c'])
    yn, stn = _affconv(Xn, None, p['conv4_n'])
    h = pl.pallas_call(
        _head_fuse_body,
        grid=(_R // _HEAD_BLK,),
        in_specs=[
            pl.BlockSpec((_HEAD_BLK, 512), lambda i: (i, 0)),
            pl.BlockSpec((2, 512), lambda i: (0, 0)),
            pl.BlockSpec((_HEAD_BLK, 512), lambda i: (i, 0)),
            pl.BlockSpec((2, 512), lambda i: (0, 0)),
        ],
        out_specs=pl.BlockSpec((_HEAD_BLK, 1024), lambda i: (i, 0)),
        out_shape=jax.ShapeDtypeStruct((_R, 1024), jnp.float32),
    )(yc, _finalize_aff(stc), yn, _finalize_aff(stn))
    y1, st1 = _affconv(h, None, p['fa'])
    y2, st2 = _affconv(y1, _finalize_aff(st1), p['pred1'])
    y3, st3 = _affconv(y2, _finalize_aff(st2), p['pred2'])
    y4, st4 = _affconv(y3, _finalize_aff(st3), p['pred3'])
    out = pl.pallas_call(
        _final_body,
        grid=(_R // _HEAD_BLK,),
        in_specs=[
            pl.BlockSpec((_HEAD_BLK, 128), lambda i: (i, 0)),
            pl.BlockSpec((2, 128), lambda i: (0, 0)),
            pl.BlockSpec((15, 128), lambda i: (0, 0)),
        ],
        out_specs=pl.BlockSpec((_HEAD_BLK, 15), lambda i: (i, 0)),
        out_shape=jax.ShapeDtypeStruct((_R, 15), jnp.float32),
    )(y4, _finalize_aff(st4), p['pred4'])
    return out.reshape(B, N, 15)
